# baseline (speedup 1.0000x reference)
import functools

import jax
import jax.numpy as jnp
from jax import lax
from jax.experimental import pallas as pl
from jax.experimental.pallas import tpu as pltpu
from jax.experimental.pallas import tpu_sc as plsc

N = 10000
NPAD = 10240
E = 320000
F = 128
CH = 128
NTILES = 32
NCHUNK = 80
EP = CH * NCHUNK
RE = E // NTILES
PADE = EP - RE
RPT = NPAD // 16

_mesh = plsc.VectorSubcoreMesh(core_axis_name="c", subcore_axis_name="s")


def _deg_body(sd_hbm, out_hbm, idx_v, ones_v, acc):
    c = lax.axis_index("c")
    s = lax.axis_index("s")
    wid = c * 16 + s
    pltpu.sync_copy(sd_hbm.at[wid], idx_v)

    @pl.loop(0, CH)
    def _zero(i):
        for j in range(F // 16):
            ones_v[i, pl.ds(j * 16, 16)] = jnp.zeros((16,), jnp.float32)

    @pl.loop(0, RPT // CH)
    def _zacc(j):
        pltpu.sync_copy(ones_v, acc.at[pl.ds(s * RPT + j * CH, CH), :])

    @pl.loop(0, CH)
    def _ones(i):
        for j in range(F // 16):
            ones_v[i, pl.ds(j * 16, 16)] = jnp.ones((16,), jnp.float32)

    plsc.subcore_barrier()

    @pl.loop(0, NCHUNK)
    def _edges(g):
        pltpu.sync_copy(ones_v, acc.at[idx_v.at[g, 1]], add=True)

    plsc.subcore_barrier()
    pltpu.sync_copy(acc.at[pl.ds(s * RPT, RPT), :],
                    out_hbm.at[c, pl.ds(s * RPT, RPT), :])


_deg_kernel = functools.partial(
    pl.kernel,
    out_type=jax.ShapeDtypeStruct((2, NPAD, F), jnp.float32),
    mesh=_mesh,
    scratch_types=[
        pltpu.VMEM((NCHUNK, 2, CH), jnp.int32),
        pltpu.VMEM((CH, F), jnp.float32),
        pltpu.VMEM_SHARED((NPAD, F), jnp.float32),
    ],
)(_deg_body)


def _prop_body(sd_hbm, z_hbm, out_hbm, idx_v, rows, acc):
    c = lax.axis_index("c")
    s = lax.axis_index("s")
    wid = c * 16 + s
    pltpu.sync_copy(sd_hbm.at[wid], idx_v)

    @pl.loop(0, CH)
    def _zero(i):
        for j in range(F // 16):
            rows[i, pl.ds(j * 16, 16)] = jnp.zeros((16,), jnp.float32)

    @pl.loop(0, RPT // CH)
    def _zacc(j):
        pltpu.sync_copy(rows, acc.at[pl.ds(s * RPT + j * CH, CH), :])

    plsc.subcore_barrier()

    @pl.loop(0, NCHUNK)
    def _edges(g):
        pltpu.sync_copy(z_hbm.at[idx_v.at[g, 0]], rows)
        pltpu.sync_copy(rows, acc.at[idx_v.at[g, 1]], add=True)

    plsc.subcore_barrier()
    pltpu.sync_copy(acc.at[pl.ds(s * RPT, RPT), :],
                    out_hbm.at[c, pl.ds(s * RPT, RPT), :])


_prop_kernel = functools.partial(
    pl.kernel,
    out_type=jax.ShapeDtypeStruct((2, NPAD, F), jnp.float32),
    mesh=_mesh,
    scratch_types=[
        pltpu.VMEM((NCHUNK, 2, CH), jnp.int32),
        pltpu.VMEM((CH, F), jnp.float32),
        pltpu.VMEM_SHARED((NPAD, F), jnp.float32),
    ],
)(_prop_body)


BLK = 2048
GRID = NPAD // BLK
DISW = 8


def _tcb_body(deg2_ref, x_ref, w1_ref, zs1_ref, dis_ref):
    dsum = deg2_ref[0, :, 0:1] + deg2_ref[1, :, 0:1] + 1.0
    dis = lax.rsqrt(dsum)
    dis_ref[...] = jnp.broadcast_to(dis, (BLK, DISW))
    zs1_ref[...] = jnp.dot(dis * x_ref[...], w1_ref[...],
                           preferred_element_type=jnp.float32)


def _tcd_body(dis_ref, p1_ref, zs1_ref, b1_ref, w2_ref, zs2_ref):
    dis = dis_ref[:, 0:1]
    accf = p1_ref[0] + p1_ref[1] + zs1_ref[...]
    h = jnp.maximum(dis * accf + b1_ref[...], 0.0)
    zs2_ref[...] = jnp.dot(dis * h, w2_ref[...],
                           preferred_element_type=jnp.float32)


def _tcf_body(dis_ref, p2_ref, zs2_ref, b2_ref, o_ref):
    dis = dis_ref[:, 0:1]
    accf = p2_ref[0] + p2_ref[1] + zs2_ref[...]
    o_ref[...] = dis * accf + b2_ref[...]


_row_spec = pl.BlockSpec((BLK, F), lambda i: (i, 0))
_pair_spec = pl.BlockSpec((2, BLK, F), lambda i: (0, i, 0))
_w_spec = pl.BlockSpec((F, F), lambda i: (0, 0))
_b_spec = pl.BlockSpec((1, F), lambda i: (0, 0))
_dis_spec = pl.BlockSpec((BLK, DISW), lambda i: (i, 0))

_row_sds = jax.ShapeDtypeStruct((NPAD, F), jnp.float32)

_tcb = pl.pallas_call(
    _tcb_body,
    grid=(GRID,),
    in_specs=[_pair_spec, _row_spec, _w_spec],
    out_specs=[_row_spec, _dis_spec],
    out_shape=[_row_sds, jax.ShapeDtypeStruct((NPAD, DISW), jnp.float32)],
)

_tcd = pl.pallas_call(
    _tcd_body,
    grid=(GRID,),
    in_specs=[_dis_spec, _pair_spec, _row_spec, _b_spec, _w_spec],
    out_specs=_row_spec,
    out_shape=_row_sds,
)

_tcf = pl.pallas_call(
    _tcf_body,
    grid=(GRID,),
    in_specs=[_dis_spec, _pair_spec, _row_spec, _b_spec],
    out_specs=_row_spec,
    out_shape=_row_sds,
)


def _pack_edges(src, dst):
    src2 = src.reshape(NTILES, RE)
    dst2 = dst.reshape(NTILES, RE)
    pad_s = jnp.zeros((NTILES, PADE), jnp.int32)
    pad_d = jnp.broadcast_to(jnp.arange(N, N + PADE, dtype=jnp.int32),
                             (NTILES, PADE))
    s_p = jnp.concatenate([src2, pad_s], axis=1).reshape(NTILES, NCHUNK, CH)
    d_p = jnp.concatenate([dst2, pad_d], axis=1).reshape(NTILES, NCHUNK, CH)
    return jnp.stack([s_p, d_p], axis=2)


def kernel(x, edge_index, W1, b1, W_mu, b_mu, W_lv, b_lv):
    src = edge_index[0].astype(jnp.int32)
    dst = edge_index[1].astype(jnp.int32)
    sd = _pack_edges(src, dst)
    x_p = jnp.pad(x, ((0, NPAD - N), (0, 0)))
    W2 = jnp.concatenate([W_mu, W_lv], axis=1)
    b1r = b1.reshape(1, F)
    b2r = jnp.concatenate([b_mu, b_lv]).reshape(1, F)

    deg2 = _deg_kernel(sd)
    zs1, disN = _tcb(deg2, x_p, W1)
    p1 = _prop_kernel(sd, zs1)
    zs2 = _tcd(disN, p1, zs1, b1r, W2)
    p2 = _prop_kernel(sd, zs2)
    o = _tcf(disN, p2, zs2, b2r)
    return o[:N, :64], o[:N, 64:]


# speedup vs baseline: 1.1232x; 1.1232x over previous
import functools

import jax
import jax.numpy as jnp
from jax import lax
from jax.experimental import pallas as pl
from jax.experimental.pallas import tpu as pltpu
from jax.experimental.pallas import tpu_sc as plsc

N = 10000
NPAD = 10240
E = 320000
F = 128
CH = 128
NTILES = 32
NCHUNK = 80
EP = CH * NCHUNK
RE = E // NTILES
PADE = EP - RE
RPT = NPAD // 16
NPH = 4
PCH = NCHUNK // NPH

_mesh = plsc.VectorSubcoreMesh(core_axis_name="c", subcore_axis_name="s")


def _deg_body(sd_hbm, out_hbm, idx_v, ones_v, acc):
    c = lax.axis_index("c")
    s = lax.axis_index("s")
    wid = c * 16 + s
    pltpu.sync_copy(sd_hbm.at[wid], idx_v)

    @pl.loop(0, CH)
    def _zero(i):
        for j in range(F // 16):
            ones_v[i, pl.ds(j * 16, 16)] = jnp.zeros((16,), jnp.float32)

    @pl.loop(0, RPT // CH)
    def _zacc(j):
        pltpu.sync_copy(ones_v, acc.at[pl.ds(s * RPT + j * CH, CH), :])

    @pl.loop(0, CH)
    def _ones(i):
        for j in range(F // 16):
            ones_v[i, pl.ds(j * 16, 16)] = jnp.ones((16,), jnp.float32)

    plsc.subcore_barrier()

    @pl.loop(0, NCHUNK)
    def _edges(g):
        pltpu.sync_copy(ones_v, acc.at[idx_v.at[g, 1]], add=True)

    plsc.subcore_barrier()
    pltpu.sync_copy(acc.at[pl.ds(s * RPT, RPT), :],
                    out_hbm.at[c, pl.ds(s * RPT, RPT), :])


_deg_kernel = functools.partial(
    pl.kernel,
    out_type=jax.ShapeDtypeStruct((2, NPAD, F), jnp.float32),
    mesh=_mesh,
    scratch_types=[
        pltpu.VMEM((NCHUNK, 2, CH), jnp.int32),
        pltpu.VMEM((CH, F), jnp.float32),
        pltpu.VMEM_SHARED((NPAD, F), jnp.float32),
    ],
)(_deg_body)


def _prop_body(sd_hbm, z_hbm, out_hbm, idx_v, rows0, rows1, acc, sem0, sem1):
    c = lax.axis_index("c")
    s = lax.axis_index("s")
    wid = c * 16 + s

    @pl.loop(0, CH)
    def _zero(i):
        for j in range(F // 16):
            rows0[i, pl.ds(j * 16, 16)] = jnp.zeros((16,), jnp.float32)

    @pl.loop(0, RPT // CH)
    def _zacc(j):
        pltpu.sync_copy(rows0, acc.at[pl.ds(s * RPT + j * CH, CH), :])

    plsc.subcore_barrier()

    for ph in range(NPH):
        pltpu.sync_copy(sd_hbm.at[wid, pl.ds(ph * PCH, PCH)], idx_v)
        pltpu.async_copy(z_hbm.at[idx_v.at[0, 0]], rows0, sem0)
        pltpu.async_copy(z_hbm.at[idx_v.at[1, 0]], rows1, sem1)

        @pl.loop(0, PCH, step=2)
        def _edges(g):
            for b, (rows, sem) in enumerate(((rows0, sem0), (rows1, sem1))):
                gg = g + b
                pltpu.make_async_copy(z_hbm.at[idx_v.at[gg, 0]], rows,
                                      sem).wait()
                pltpu.sync_copy(rows, acc.at[idx_v.at[gg, 1]], add=True)

                @pl.when(gg + 2 < PCH)
                def _next():
                    pltpu.async_copy(z_hbm.at[idx_v.at[gg + 2, 0]], rows, sem)

    plsc.subcore_barrier()
    pltpu.sync_copy(acc.at[pl.ds(s * RPT, RPT), :],
                    out_hbm.at[c, pl.ds(s * RPT, RPT), :])


_prop_kernel = functools.partial(
    pl.kernel,
    out_type=jax.ShapeDtypeStruct((2, NPAD, F), jnp.float32),
    mesh=_mesh,
    scratch_types=[
        pltpu.VMEM((PCH, 2, CH), jnp.int32),
        pltpu.VMEM((CH, F), jnp.float32),
        pltpu.VMEM((CH, F), jnp.float32),
        pltpu.VMEM_SHARED((NPAD, F), jnp.float32),
        pltpu.SemaphoreType.DMA,
        pltpu.SemaphoreType.DMA,
    ],
)(_prop_body)


BLK = 2048
GRID = NPAD // BLK
DISW = 8


def _tcb_body(deg2_ref, x_ref, w1_ref, zs1_ref, dis_ref):
    dsum = deg2_ref[0, :, 0:1] + deg2_ref[1, :, 0:1] + 1.0
    dis = lax.rsqrt(dsum)
    dis_ref[...] = jnp.broadcast_to(dis, (BLK, DISW))
    zs1_ref[...] = jnp.dot(dis * x_ref[...], w1_ref[...],
                           preferred_element_type=jnp.float32)


def _tcd_body(dis_ref, p1_ref, zs1_ref, b1_ref, w2_ref, zs2_ref):
    dis = dis_ref[:, 0:1]
    accf = p1_ref[0] + p1_ref[1] + zs1_ref[...]
    h = jnp.maximum(dis * accf + b1_ref[...], 0.0)
    zs2_ref[...] = jnp.dot(dis * h, w2_ref[...],
                           preferred_element_type=jnp.float32)


def _tcf_body(dis_ref, p2_ref, zs2_ref, b2_ref, o_ref):
    dis = dis_ref[:, 0:1]
    accf = p2_ref[0] + p2_ref[1] + zs2_ref[...]
    o_ref[...] = dis * accf + b2_ref[...]


_row_spec = pl.BlockSpec((BLK, F), lambda i: (i, 0))
_pair_spec = pl.BlockSpec((2, BLK, F), lambda i: (0, i, 0))
_w_spec = pl.BlockSpec((F, F), lambda i: (0, 0))
_b_spec = pl.BlockSpec((1, F), lambda i: (0, 0))
_dis_spec = pl.BlockSpec((BLK, DISW), lambda i: (i, 0))

_row_sds = jax.ShapeDtypeStruct((NPAD, F), jnp.float32)

_tcb = pl.pallas_call(
    _tcb_body,
    grid=(GRID,),
    in_specs=[_pair_spec, _row_spec, _w_spec],
    out_specs=[_row_spec, _dis_spec],
    out_shape=[_row_sds, jax.ShapeDtypeStruct((NPAD, DISW), jnp.float32)],
)

_tcd = pl.pallas_call(
    _tcd_body,
    grid=(GRID,),
    in_specs=[_dis_spec, _pair_spec, _row_spec, _b_spec, _w_spec],
    out_specs=_row_spec,
    out_shape=_row_sds,
)

_tcf = pl.pallas_call(
    _tcf_body,
    grid=(GRID,),
    in_specs=[_dis_spec, _pair_spec, _row_spec, _b_spec],
    out_specs=_row_spec,
    out_shape=_row_sds,
)


def _pack_edges(src, dst):
    src2 = src.reshape(NTILES, RE)
    dst2 = dst.reshape(NTILES, RE)
    pad_s = jnp.zeros((NTILES, PADE), jnp.int32)
    pad_d = jnp.broadcast_to(jnp.arange(N, N + PADE, dtype=jnp.int32),
                             (NTILES, PADE))
    s_p = jnp.concatenate([src2, pad_s], axis=1).reshape(NTILES, NCHUNK, CH)
    d_p = jnp.concatenate([dst2, pad_d], axis=1).reshape(NTILES, NCHUNK, CH)
    return jnp.stack([s_p, d_p], axis=2)


def kernel(x, edge_index, W1, b1, W_mu, b_mu, W_lv, b_lv):
    src = edge_index[0].astype(jnp.int32)
    dst = edge_index[1].astype(jnp.int32)
    sd = _pack_edges(src, dst)
    x_p = jnp.pad(x, ((0, NPAD - N), (0, 0)))
    W2 = jnp.concatenate([W_mu, W_lv], axis=1)
    b1r = b1.reshape(1, F)
    b2r = jnp.concatenate([b_mu, b_lv]).reshape(1, F)

    deg2 = _deg_kernel(sd)
    zs1, disN = _tcb(deg2, x_p, W1)
    p1 = _prop_kernel(sd, zs1)
    zs2 = _tcd(disN, p1, zs1, b1r, W2)
    p2 = _prop_kernel(sd, zs2)
    o = _tcf(disN, p2, zs2, b2r)
    return o[:N, :64], o[:N, 64:]


# speedup vs baseline: 1.1241x; 1.0007x over previous
import functools

import jax
import jax.numpy as jnp
from jax import lax
from jax.experimental import pallas as pl
from jax.experimental.pallas import tpu as pltpu
from jax.experimental.pallas import tpu_sc as plsc

N = 10000
NPAD = 10240
E = 320000
F = 128
CH = 128
NTILES = 32
NCHUNK = 80
EP = CH * NCHUNK
RE = E // NTILES
PADE = EP - RE
RPT = NPAD // 16
NPH = 4
PCH = NCHUNK // NPH

_mesh = plsc.VectorSubcoreMesh(core_axis_name="c", subcore_axis_name="s")


def _deg_body(sd_hbm, out_hbm, idx_v, ones_v, acc):
    c = lax.axis_index("c")
    s = lax.axis_index("s")
    wid = c * 16 + s
    pltpu.sync_copy(sd_hbm.at[wid], idx_v)

    @pl.loop(0, CH)
    def _zero(i):
        for j in range(F // 16):
            ones_v[i, pl.ds(j * 16, 16)] = jnp.zeros((16,), jnp.float32)

    @pl.loop(0, RPT // CH)
    def _zacc(j):
        pltpu.sync_copy(ones_v, acc.at[pl.ds(s * RPT + j * CH, CH), :])

    @pl.loop(0, CH)
    def _ones(i):
        for j in range(F // 16):
            ones_v[i, pl.ds(j * 16, 16)] = jnp.ones((16,), jnp.float32)

    plsc.subcore_barrier()

    @pl.loop(0, NCHUNK)
    def _edges(g):
        pltpu.sync_copy(ones_v, acc.at[idx_v.at[g, 1]], add=True)

    plsc.subcore_barrier()
    pltpu.sync_copy(acc.at[pl.ds(s * RPT, RPT), :],
                    out_hbm.at[c, pl.ds(s * RPT, RPT), :])


_deg_kernel = functools.partial(
    pl.kernel,
    out_type=jax.ShapeDtypeStruct((2, NPAD, F), jnp.float32),
    mesh=_mesh,
    scratch_types=[
        pltpu.VMEM((NCHUNK, 2, CH), jnp.int32),
        pltpu.VMEM((CH, F), jnp.float32),
        pltpu.VMEM_SHARED((NPAD, F), jnp.float32),
    ],
)(_deg_body)


def _prop_body(sd_hbm, z_hbm, out_hbm, idx_v, rows0, rows1, acc, sem0, sem1):
    c = lax.axis_index("c")
    s = lax.axis_index("s")
    wid = c * 16 + s

    @pl.loop(0, CH)
    def _zero(i):
        for j in range(F // 16):
            rows0[i, pl.ds(j * 16, 16)] = jnp.zeros((16,), jnp.float32)

    @pl.loop(0, RPT // CH)
    def _zacc(j):
        pltpu.sync_copy(rows0, acc.at[pl.ds(s * RPT + j * CH, CH), :])

    plsc.subcore_barrier()

    for ph in range(NPH):
        pltpu.sync_copy(sd_hbm.at[wid, pl.ds(ph * PCH, PCH)], idx_v)
        pltpu.async_copy(z_hbm.at[idx_v.at[0, 0]], rows0, sem0)
        pltpu.async_copy(z_hbm.at[idx_v.at[1, 0]], rows1, sem1)

        @pl.loop(0, PCH, step=2)
        def _edges(g):
            for b, (rows, sem) in enumerate(((rows0, sem0), (rows1, sem1))):
                gg = g + b
                pltpu.make_async_copy(z_hbm.at[idx_v.at[gg, 0]], rows,
                                      sem).wait()
                pltpu.sync_copy(rows, acc.at[idx_v.at[gg, 1]], add=True)

                @pl.when(gg + 2 < PCH)
                def _next():
                    pltpu.async_copy(z_hbm.at[idx_v.at[gg + 2, 0]], rows, sem)

    plsc.subcore_barrier()
    pltpu.sync_copy(acc.at[pl.ds(s * RPT, RPT), :],
                    out_hbm.at[c, pl.ds(s * RPT, RPT), :])


_prop_kernel = functools.partial(
    pl.kernel,
    out_type=jax.ShapeDtypeStruct((2, NPAD, F), jnp.float32),
    mesh=_mesh,
    scratch_types=[
        pltpu.VMEM((PCH, 2, CH), jnp.int32),
        pltpu.VMEM((CH, F), jnp.float32),
        pltpu.VMEM((CH, F), jnp.float32),
        pltpu.VMEM_SHARED((NPAD, F), jnp.float32),
        pltpu.SemaphoreType.DMA,
        pltpu.SemaphoreType.DMA,
    ],
)(_prop_body)


BLK = 2048
GRID = NPAD // BLK
DISW = 8


def _tcm_body(x_ref, w1_ref, y1_ref):
    y1_ref[...] = jnp.dot(x_ref[...], w1_ref[...],
                          preferred_element_type=jnp.float32)


def _tcb_body(deg2_ref, y1_ref, zs1_ref, dis_ref):
    dsum = deg2_ref[0, :, 0:1] + deg2_ref[1, :, 0:1] + 1.0
    dis = lax.rsqrt(dsum)
    dis_ref[...] = jnp.broadcast_to(dis, (BLK, DISW))
    zs1_ref[...] = dis * y1_ref[...]


def _tcd_body(dis_ref, p1_ref, zs1_ref, b1_ref, w2_ref, zs2_ref):
    dis = dis_ref[:, 0:1]
    accf = p1_ref[0] + p1_ref[1] + zs1_ref[...]
    h = jnp.maximum(dis * accf + b1_ref[...], 0.0)
    zs2_ref[...] = jnp.dot(dis * h, w2_ref[...],
                           preferred_element_type=jnp.float32)


def _tcf_body(dis_ref, p2_ref, zs2_ref, b2_ref, o_ref):
    dis = dis_ref[:, 0:1]
    accf = p2_ref[0] + p2_ref[1] + zs2_ref[...]
    o_ref[...] = dis * accf + b2_ref[...]


_row_spec = pl.BlockSpec((BLK, F), lambda i: (i, 0))
_pair_spec = pl.BlockSpec((2, BLK, F), lambda i: (0, i, 0))
_w_spec = pl.BlockSpec((F, F), lambda i: (0, 0))
_b_spec = pl.BlockSpec((1, F), lambda i: (0, 0))
_dis_spec = pl.BlockSpec((BLK, DISW), lambda i: (i, 0))

_row_sds = jax.ShapeDtypeStruct((NPAD, F), jnp.float32)

_tcm = pl.pallas_call(
    _tcm_body,
    grid=(GRID,),
    in_specs=[_row_spec, _w_spec],
    out_specs=_row_spec,
    out_shape=_row_sds,
)

_tcb = pl.pallas_call(
    _tcb_body,
    grid=(GRID,),
    in_specs=[_pair_spec, _row_spec],
    out_specs=[_row_spec, _dis_spec],
    out_shape=[_row_sds, jax.ShapeDtypeStruct((NPAD, DISW), jnp.float32)],
)

_tcd = pl.pallas_call(
    _tcd_body,
    grid=(GRID,),
    in_specs=[_dis_spec, _pair_spec, _row_spec, _b_spec, _w_spec],
    out_specs=_row_spec,
    out_shape=_row_sds,
)

_tcf = pl.pallas_call(
    _tcf_body,
    grid=(GRID,),
    in_specs=[_dis_spec, _pair_spec, _row_spec, _b_spec],
    out_specs=_row_spec,
    out_shape=_row_sds,
)


def _pack_edges(src, dst):
    src2 = src.reshape(NTILES, RE)
    dst2 = dst.reshape(NTILES, RE)
    pad_s = jnp.zeros((NTILES, PADE), jnp.int32)
    pad_d = jnp.broadcast_to(jnp.arange(N, N + PADE, dtype=jnp.int32),
                             (NTILES, PADE))
    s_p = jnp.concatenate([src2, pad_s], axis=1).reshape(NTILES, NCHUNK, CH)
    d_p = jnp.concatenate([dst2, pad_d], axis=1).reshape(NTILES, NCHUNK, CH)
    return jnp.stack([s_p, d_p], axis=2)


def kernel(x, edge_index, W1, b1, W_mu, b_mu, W_lv, b_lv):
    src = edge_index[0].astype(jnp.int32)
    dst = edge_index[1].astype(jnp.int32)
    sd = _pack_edges(src, dst)
    x_p = jnp.pad(x, ((0, NPAD - N), (0, 0)))
    W2 = jnp.concatenate([W_mu, W_lv], axis=1)
    b1r = b1.reshape(1, F)
    b2r = jnp.concatenate([b_mu, b_lv]).reshape(1, F)

    y1 = _tcm(x_p, W1)
    deg2 = _deg_kernel(sd)
    zs1, disN = _tcb(deg2, y1)
    p1 = _prop_kernel(sd, zs1)
    zs2 = _tcd(disN, p1, zs1, b1r, W2)
    p2 = _prop_kernel(sd, zs2)
    o = _tcf(disN, p2, zs2, b2r)
    return o[:N, :64], o[:N, 64:]


# speedup vs baseline: 1.1246x; 1.0005x over previous
import functools

import jax
import jax.numpy as jnp
from jax import lax
from jax.experimental import pallas as pl
from jax.experimental.pallas import tpu as pltpu
from jax.experimental.pallas import tpu_sc as plsc

N = 10000
NPAD = 10240
E = 320000
F = 128
CH = 128
NTILES = 32
NCHUNK = 80
EP = CH * NCHUNK
RE = E // NTILES
PADE = EP - RE
RPT = NPAD // 16
NPH = 4
PCH = NCHUNK // NPH

_mesh = plsc.VectorSubcoreMesh(core_axis_name="c", subcore_axis_name="s")


def _deg_body(sd_hbm, out_hbm, idx_v, ones_v, acc, dsem):
    c = lax.axis_index("c")
    s = lax.axis_index("s")
    wid = c * 16 + s
    pltpu.sync_copy(sd_hbm.at[wid], idx_v)

    @pl.loop(0, CH)
    def _zero(i):
        for j in range(F // 16):
            ones_v[i, pl.ds(j * 16, 16)] = jnp.zeros((16,), jnp.float32)

    @pl.loop(0, RPT // CH)
    def _zacc(j):
        pltpu.sync_copy(ones_v, acc.at[pl.ds(s * RPT + j * CH, CH), :])

    @pl.loop(0, CH)
    def _ones(i):
        for j in range(F // 16):
            ones_v[i, pl.ds(j * 16, 16)] = jnp.ones((16,), jnp.float32)

    plsc.subcore_barrier()

    @pl.loop(0, NCHUNK, step=8)
    def _edges(g):
        for b in range(8):
            pltpu.async_copy(ones_v, acc.at[idx_v.at[g + b, 1]], dsem,
                             add=True)
        for b in range(8):
            pltpu.make_async_copy(ones_v, acc.at[idx_v.at[g + b, 1]],
                                  dsem).wait()

    plsc.subcore_barrier()
    pltpu.sync_copy(acc.at[pl.ds(s * RPT, RPT), :],
                    out_hbm.at[c, pl.ds(s * RPT, RPT), :])


_deg_kernel = functools.partial(
    pl.kernel,
    out_type=jax.ShapeDtypeStruct((2, NPAD, F), jnp.float32),
    mesh=_mesh,
    scratch_types=[
        pltpu.VMEM((NCHUNK, 2, CH), jnp.int32),
        pltpu.VMEM((CH, F), jnp.float32),
        pltpu.VMEM_SHARED((NPAD, F), jnp.float32),
        pltpu.SemaphoreType.DMA,
    ],
)(_deg_body)


def _prop_body(sd_hbm, z_hbm, out_hbm, idx_v, rows0, rows1, acc, sem0, sem1):
    c = lax.axis_index("c")
    s = lax.axis_index("s")
    wid = c * 16 + s

    @pl.loop(0, CH)
    def _zero(i):
        for j in range(F // 16):
            rows0[i, pl.ds(j * 16, 16)] = jnp.zeros((16,), jnp.float32)

    @pl.loop(0, RPT // CH)
    def _zacc(j):
        pltpu.sync_copy(rows0, acc.at[pl.ds(s * RPT + j * CH, CH), :])

    plsc.subcore_barrier()

    for ph in range(NPH):
        pltpu.sync_copy(sd_hbm.at[wid, pl.ds(ph * PCH, PCH)], idx_v)
        pltpu.async_copy(z_hbm.at[idx_v.at[0, 0]], rows0, sem0)
        pltpu.async_copy(z_hbm.at[idx_v.at[1, 0]], rows1, sem1)

        @pl.loop(0, PCH, step=2)
        def _edges(g):
            for b, (rows, sem) in enumerate(((rows0, sem0), (rows1, sem1))):
                gg = g + b
                pltpu.make_async_copy(z_hbm.at[idx_v.at[gg, 0]], rows,
                                      sem).wait()
                pltpu.sync_copy(rows, acc.at[idx_v.at[gg, 1]], add=True)

                @pl.when(gg + 2 < PCH)
                def _next():
                    pltpu.async_copy(z_hbm.at[idx_v.at[gg + 2, 0]], rows, sem)

    plsc.subcore_barrier()
    pltpu.sync_copy(acc.at[pl.ds(s * RPT, RPT), :],
                    out_hbm.at[c, pl.ds(s * RPT, RPT), :])


_prop_kernel = functools.partial(
    pl.kernel,
    out_type=jax.ShapeDtypeStruct((2, NPAD, F), jnp.float32),
    mesh=_mesh,
    scratch_types=[
        pltpu.VMEM((PCH, 2, CH), jnp.int32),
        pltpu.VMEM((CH, F), jnp.float32),
        pltpu.VMEM((CH, F), jnp.float32),
        pltpu.VMEM_SHARED((NPAD, F), jnp.float32),
        pltpu.SemaphoreType.DMA,
        pltpu.SemaphoreType.DMA,
    ],
)(_prop_body)


BLK = 2048
GRID = NPAD // BLK
DISW = 8


def _tcm_body(x_ref, w1_ref, y1_ref):
    y1_ref[...] = jnp.dot(x_ref[...], w1_ref[...],
                          preferred_element_type=jnp.float32)


def _tcb_body(deg2_ref, y1_ref, zs1_ref, dis_ref):
    dsum = deg2_ref[0, :, 0:1] + deg2_ref[1, :, 0:1] + 1.0
    dis = lax.rsqrt(dsum)
    dis_ref[...] = jnp.broadcast_to(dis, (BLK, DISW))
    zs1_ref[...] = dis * y1_ref[...]


def _tcd_body(dis_ref, p1_ref, zs1_ref, b1_ref, w2_ref, zs2_ref):
    dis = dis_ref[:, 0:1]
    accf = p1_ref[0] + p1_ref[1] + zs1_ref[...]
    h = jnp.maximum(dis * accf + b1_ref[...], 0.0)
    zs2_ref[...] = jnp.dot(dis * h, w2_ref[...],
                           preferred_element_type=jnp.float32)


def _tcf_body(dis_ref, p2_ref, zs2_ref, b2_ref, o_ref):
    dis = dis_ref[:, 0:1]
    accf = p2_ref[0] + p2_ref[1] + zs2_ref[...]
    o_ref[...] = dis * accf + b2_ref[...]


_row_spec = pl.BlockSpec((BLK, F), lambda i: (i, 0))
_pair_spec = pl.BlockSpec((2, BLK, F), lambda i: (0, i, 0))
_w_spec = pl.BlockSpec((F, F), lambda i: (0, 0))
_b_spec = pl.BlockSpec((1, F), lambda i: (0, 0))
_dis_spec = pl.BlockSpec((BLK, DISW), lambda i: (i, 0))

_row_sds = jax.ShapeDtypeStruct((NPAD, F), jnp.float32)

_tcm = pl.pallas_call(
    _tcm_body,
    grid=(GRID,),
    in_specs=[_row_spec, _w_spec],
    out_specs=_row_spec,
    out_shape=_row_sds,
)

_tcb = pl.pallas_call(
    _tcb_body,
    grid=(GRID,),
    in_specs=[_pair_spec, _row_spec],
    out_specs=[_row_spec, _dis_spec],
    out_shape=[_row_sds, jax.ShapeDtypeStruct((NPAD, DISW), jnp.float32)],
)

_tcd = pl.pallas_call(
    _tcd_body,
    grid=(GRID,),
    in_specs=[_dis_spec, _pair_spec, _row_spec, _b_spec, _w_spec],
    out_specs=_row_spec,
    out_shape=_row_sds,
)

_tcf = pl.pallas_call(
    _tcf_body,
    grid=(GRID,),
    in_specs=[_dis_spec, _pair_spec, _row_spec, _b_spec],
    out_specs=_row_spec,
    out_shape=_row_sds,
)


def _pack_edges(src, dst):
    src2 = src.reshape(NTILES, RE)
    dst2 = dst.reshape(NTILES, RE)
    pad_s = jnp.zeros((NTILES, PADE), jnp.int32)
    pad_d = jnp.broadcast_to(jnp.arange(N, N + PADE, dtype=jnp.int32),
                             (NTILES, PADE))
    s_p = jnp.concatenate([src2, pad_s], axis=1).reshape(NTILES, NCHUNK, CH)
    d_p = jnp.concatenate([dst2, pad_d], axis=1).reshape(NTILES, NCHUNK, CH)
    return jnp.stack([s_p, d_p], axis=2)


def kernel(x, edge_index, W1, b1, W_mu, b_mu, W_lv, b_lv):
    src = edge_index[0].astype(jnp.int32)
    dst = edge_index[1].astype(jnp.int32)
    sd = _pack_edges(src, dst)
    x_p = jnp.pad(x, ((0, NPAD - N), (0, 0)))
    W2 = jnp.concatenate([W_mu, W_lv], axis=1)
    b1r = b1.reshape(1, F)
    b2r = jnp.concatenate([b_mu, b_lv]).reshape(1, F)

    y1 = _tcm(x_p, W1)
    deg2 = _deg_kernel(sd)
    zs1, disN = _tcb(deg2, y1)
    p1 = _prop_kernel(sd, zs1)
    zs2 = _tcd(disN, p1, zs1, b1r, W2)
    p2 = _prop_kernel(sd, zs2)
    o = _tcf(disN, p2, zs2, b2r)
    return o[:N, :64], o[:N, 64:]
